# baseline (speedup 1.0000x reference)
import functools

import jax
import jax.numpy as jnp
from jax import lax
from jax.experimental import pallas as pl
from jax.experimental.pallas import tpu as pltpu
from jax.experimental.pallas import tpu_sc as plsc

B = 16384
EMB = 64
NC = 2
NS = 16
NW = NC * NS
BPW = B // NW
CH = 256
LANES = 16


@functools.lru_cache(maxsize=None)
def _make_sc_gather():
    @functools.partial(
        pl.kernel,
        mesh=plsc.VectorSubcoreMesh(core_axis_name="c", subcore_axis_name="s"),
        compiler_params=pltpu.CompilerParams(use_tc_tiling_on_sc=False),
        out_type=(
            jax.ShapeDtypeStruct((B, EMB), jnp.float32),
            jax.ShapeDtypeStruct((B, EMB), jnp.float32),
            jax.ShapeDtypeStruct((B, EMB), jnp.float32),
        ),
        scratch_types=(
            pltpu.VMEM((BPW,), jnp.int32),
            pltpu.VMEM((BPW,), jnp.int32),
            pltpu.VMEM((CH, EMB), jnp.float32),
            pltpu.VMEM((CH, EMB), jnp.float32),
            pltpu.VMEM((CH, EMB), jnp.float32),
            pltpu.VMEM((CH, EMB), jnp.float32),
            pltpu.SemaphoreType.DMA,
            pltpu.SemaphoreType.DMA,
            pltpu.SemaphoreType.DMA,
            pltpu.SemaphoreType.DMA,
        ),
    )
    def _sc_gather(users, items, gu_t, gi_t, mu_t, mi_t,
                   gmf_out, mu_out, mi_out,
                   uidx, iidx, gu, gi, mu, mi, s0, s1, s2, s3):
        wid = lax.axis_index("s") * NC + lax.axis_index("c")
        base = wid * BPW
        pltpu.sync_copy(users.at[pl.ds(base, BPW)], uidx)
        pltpu.sync_copy(items.at[pl.ds(base, BPW)], iidx)
        for c in range(BPW // CH):
            off = c * CH
            usl = uidx.at[pl.ds(off, CH)]
            isl = iidx.at[pl.ds(off, CH)]
            cg0 = pltpu.async_copy(gu_t.at[usl], gu, s0)
            cg1 = pltpu.async_copy(gi_t.at[isl], gi, s1)
            cm0 = pltpu.async_copy(mu_t.at[usl], mu, s2)
            cm1 = pltpu.async_copy(mi_t.at[isl], mi, s3)
            cg0.wait()
            cg1.wait()

            def _prod(r, carry):
                for j in range(EMB // LANES):
                    sl = pl.ds(j * LANES, LANES)
                    gu[r, sl] = gu[r, sl] * gi[r, sl]
                return carry

            lax.fori_loop(0, CH, _prod, 0)
            pltpu.sync_copy(gu, gmf_out.at[pl.ds(base + off, CH)])
            cm0.wait()
            pltpu.sync_copy(mu, mu_out.at[pl.ds(base + off, CH)])
            cm1.wait()
            pltpu.sync_copy(mi, mi_out.at[pl.ds(base + off, CH)])

    return _sc_gather


BK = 2048


def _mlp_body(gmf_ref, mu_ref, mi_ref, w1_ref, b1_ref, w2_ref, b2_ref,
              wp_ref, bp_ref, out_ref):
    x = jnp.concatenate([mu_ref[...], mi_ref[...]], axis=1)
    h = jnp.dot(x, w1_ref[...], preferred_element_type=jnp.float32)
    h = jnp.maximum(h + b1_ref[...], 0.0)
    h = jnp.dot(h, w2_ref[...], preferred_element_type=jnp.float32)
    h = jnp.maximum(h + b2_ref[...], 0.0)
    cat = jnp.concatenate([gmf_ref[...], h], axis=1)
    pred = jnp.dot(cat, wp_ref[...], preferred_element_type=jnp.float32)
    out_ref[...] = pred[:, 0] + bp_ref[0, 0]


def kernel(users, items, gmf_user_table, gmf_item_table, mlp_user_table,
           mlp_item_table, W1, b1, W2, b2, Wp, bp):
    users = users.astype(jnp.int32)
    items = items.astype(jnp.int32)
    gmf_prod, mu_rows, mi_rows = _make_sc_gather()(
        users, items, gmf_user_table, gmf_item_table,
        mlp_user_table, mlp_item_table)

    grid = B // BK
    pred = pl.pallas_call(
        _mlp_body,
        grid=(grid,),
        in_specs=[
            pl.BlockSpec((BK, EMB), lambda i: (i, 0)),
            pl.BlockSpec((BK, EMB), lambda i: (i, 0)),
            pl.BlockSpec((BK, EMB), lambda i: (i, 0)),
            pl.BlockSpec((2 * EMB, 128), lambda i: (0, 0)),
            pl.BlockSpec((1, 128), lambda i: (0, 0)),
            pl.BlockSpec((128, EMB), lambda i: (0, 0)),
            pl.BlockSpec((1, EMB), lambda i: (0, 0)),
            pl.BlockSpec((2 * EMB, 1), lambda i: (0, 0)),
            pl.BlockSpec((1, 1), lambda i: (0, 0)),
        ],
        out_specs=pl.BlockSpec((BK,), lambda i: (i,)),
        out_shape=jax.ShapeDtypeStruct((B,), jnp.float32),
    )(gmf_prod, mu_rows, mi_rows, W1, b1.reshape(1, 128), W2,
      b2.reshape(1, EMB), Wp, bp.reshape(1, 1))
    return pred


# speedup vs baseline: 2.6294x; 2.6294x over previous
import functools

import jax
import jax.numpy as jnp
from jax import lax
from jax.experimental import pallas as pl
from jax.experimental.pallas import tpu as pltpu
from jax.experimental.pallas import tpu_sc as plsc

B = 16384
EMB = 64
D2 = 2 * EMB
NV = 1000000
NC = 2
NS = 16
NW = NC * NS
BPW = B // NW
CH = 256

NB = 2048


def _relayout_body(gu_ref, mu_ref, gi_ref, mi_ref, u_out, i_out):
    u = jnp.concatenate([gu_ref[...], mu_ref[...]], axis=0)
    i = jnp.concatenate([gi_ref[...], mi_ref[...]], axis=0)
    u_out[...] = u.T
    i_out[...] = i.T


def _relayout(guT, muT, giT, miT):
    grid = (NV + NB - 1) // NB
    return pl.pallas_call(
        _relayout_body,
        grid=(grid,),
        in_specs=[pl.BlockSpec((EMB, NB), lambda k: (0, k))] * 4,
        out_specs=[pl.BlockSpec((NB, D2), lambda k: (k, 0))] * 2,
        out_shape=[jax.ShapeDtypeStruct((NV, D2), jnp.float32)] * 2,
    )(guT, muT, giT, miT)


@functools.lru_cache(maxsize=None)
def _make_sc_gather():
    @functools.partial(
        pl.kernel,
        mesh=plsc.VectorSubcoreMesh(core_axis_name="c", subcore_axis_name="s"),
        out_type=(
            jax.ShapeDtypeStruct((B, D2), jnp.float32),
            jax.ShapeDtypeStruct((B, D2), jnp.float32),
        ),
        scratch_types=(
            pltpu.VMEM((BPW,), jnp.int32),
            pltpu.VMEM((BPW,), jnp.int32),
            pltpu.VMEM((CH, D2), jnp.float32),
            pltpu.VMEM((CH, D2), jnp.float32),
            pltpu.SemaphoreType.DMA,
            pltpu.SemaphoreType.DMA,
        ),
    )
    def _sc_gather(users, items, ut, it, u_out, i_out,
                   uidx, iidx, ubuf, ibuf, s0, s1):
        wid = lax.axis_index("s") * NC + lax.axis_index("c")
        base = wid * BPW
        pltpu.sync_copy(users.at[pl.ds(base, BPW)], uidx)
        pltpu.sync_copy(items.at[pl.ds(base, BPW)], iidx)
        for c in range(BPW // CH):
            off = c * CH
            cu = pltpu.async_copy(ut.at[uidx.at[pl.ds(off, CH)]], ubuf, s0)
            ci = pltpu.async_copy(it.at[iidx.at[pl.ds(off, CH)]], ibuf, s1)
            cu.wait()
            pltpu.sync_copy(ubuf, u_out.at[pl.ds(base + off, CH)])
            ci.wait()
            pltpu.sync_copy(ibuf, i_out.at[pl.ds(base + off, CH)])

    return _sc_gather


BK = 2048


def _mlp_body(u_ref, i_ref, w1_ref, b1_ref, w2_ref, b2_ref,
              wp_ref, bp_ref, out_ref):
    u = u_ref[...]
    i = i_ref[...]
    gmf = u[:, :EMB] * i[:, :EMB]
    x = jnp.concatenate([u[:, EMB:], i[:, EMB:]], axis=1)
    h = jnp.dot(x, w1_ref[...], preferred_element_type=jnp.float32)
    h = jnp.maximum(h + b1_ref[...], 0.0)
    h = jnp.dot(h, w2_ref[...], preferred_element_type=jnp.float32)
    h = jnp.maximum(h + b2_ref[...], 0.0)
    cat = jnp.concatenate([gmf, h], axis=1)
    pred = jnp.dot(cat, wp_ref[...], preferred_element_type=jnp.float32)
    out_ref[...] = pred[:, 0] + bp_ref[0, 0]


def kernel(users, items, gmf_user_table, gmf_item_table, mlp_user_table,
           mlp_item_table, W1, b1, W2, b2, Wp, bp):
    users = users.astype(jnp.int32)
    items = items.astype(jnp.int32)
    ut, it = _relayout(gmf_user_table.T, mlp_user_table.T,
                       gmf_item_table.T, mlp_item_table.T)
    u_rows, i_rows = _make_sc_gather()(users, items, ut, it)

    grid = B // BK
    pred = pl.pallas_call(
        _mlp_body,
        grid=(grid,),
        in_specs=[
            pl.BlockSpec((BK, D2), lambda i: (i, 0)),
            pl.BlockSpec((BK, D2), lambda i: (i, 0)),
            pl.BlockSpec((D2, 128), lambda i: (0, 0)),
            pl.BlockSpec((1, 128), lambda i: (0, 0)),
            pl.BlockSpec((128, EMB), lambda i: (0, 0)),
            pl.BlockSpec((1, EMB), lambda i: (0, 0)),
            pl.BlockSpec((D2, 1), lambda i: (0, 0)),
            pl.BlockSpec((1, 1), lambda i: (0, 0)),
        ],
        out_specs=pl.BlockSpec((BK,), lambda i: (i,)),
        out_shape=jax.ShapeDtypeStruct((B,), jnp.float32),
    )(u_rows, i_rows, W1, b1.reshape(1, 128), W2,
      b2.reshape(1, EMB), Wp, bp.reshape(1, 1))
    return pred
